# baseline (speedup 1.0000x reference)
import functools

import jax
import jax.numpy as jnp
from jax import lax
from jax.experimental import pallas as pl
from jax.experimental.pallas import tpu as pltpu
from jax.experimental.pallas import tpu_sc as plsc

EMBD = 128
SEQ = 50
LANES = 16
NCORES = 2
NSUB = 16
NW = NCORES * NSUB
G2 = EMBD // (2 * LANES)
WPR = EMBD // 2
RBIAS = jnp.int32(0x8000)
HIMASK = jnp.int32(-65536)


def _sc_pool(emb, idx_flat, n_samples):
    vocab = emb.shape[0]
    per_w = n_samples // NW
    CH = 8
    PAIRS = per_w // (2 * CH)
    UN = 5
    CROWS = 60
    rows_per_tile = vocab // NSUB
    tail = vocab - rows_per_tile * NSUB
    csteps = rows_per_tile // CROWS
    assert rows_per_tile % CROWS == 0
    mesh = plsc.VectorSubcoreMesh(core_axis_name="c", subcore_axis_name="s")

    @functools.partial(
        pl.kernel,
        mesh=mesh,
        out_type=(
            jax.ShapeDtypeStruct((n_samples, EMBD), jnp.float32),
            jax.ShapeDtypeStruct((vocab, WPR), jnp.int32),
            jax.ShapeDtypeStruct((vocab, WPR), jnp.int32),
        ),
        scratch_types=[
            pltpu.VMEM((CROWS, EMBD), jnp.float32),
            pltpu.VMEM((CROWS, WPR), jnp.int32),
            pltpu.VMEM((per_w * SEQ,), jnp.int32),
            pltpu.VMEM((CH * SEQ, WPR), jnp.int32),
            pltpu.VMEM((CH * SEQ, WPR), jnp.int32),
            pltpu.VMEM((CH, EMBD), jnp.float32),
            pltpu.VMEM((CH, EMBD), jnp.float32),
            pltpu.SemaphoreType.DMA,
            pltpu.SemaphoreType.DMA,
            pltpu.SemaphoreType.DMA,
            pltpu.SemaphoreType.DMA,
        ],
        compiler_params=pltpu.CompilerParams(use_tc_tiling_on_sc=False),
    )
    def pool_kernel(emb_hbm, idx_hbm, out_hbm, embc0, embc1,
                    fin, fout, idx_v, rows0, rows1, acc0, acc1,
                    g0, g1, o0, o1):
        cid = lax.axis_index("c")
        sid = lax.axis_index("s")
        wid = sid * NCORES + cid
        base = wid * per_w
        pltpu.sync_copy(idx_hbm.at[pl.ds(base * SEQ, per_w * SEQ)], idx_v)

        def pack_rows(nrows, embc, row0):
            pltpu.sync_copy(emb_hbm.at[pl.ds(row0, nrows)],
                            fin.at[pl.ds(0, nrows)])

            def rbody(r, carry):
                for g in range(G2):
                    e = lax.bitcast_convert_type(
                        fin[r, pl.ds(32 * g, LANES)], jnp.int32)
                    o = lax.bitcast_convert_type(
                        fin[r, pl.ds(32 * g + LANES, LANES)], jnp.int32)
                    lo = lax.shift_right_logical(e + RBIAS, 16)
                    hi = (o + RBIAS) & HIMASK
                    fout[r, pl.ds(LANES * g, LANES)] = hi | lo
                return carry

            lax.fori_loop(0, nrows, rbody, 0)
            pltpu.sync_copy(fout.at[pl.ds(0, nrows)],
                            embc.at[pl.ds(row0, nrows)])

        def convert(embc):
            tile_r0 = sid * rows_per_tile

            def cbody(k, carry):
                pack_rows(CROWS, embc, tile_r0 + k * CROWS)
                return carry

            lax.fori_loop(0, csteps, cbody, 0)

            @pl.when(sid == 0)
            def _():
                pack_rows(tail, embc, vocab - tail)

        @pl.when(cid == 0)
        def _():
            convert(embc0)

        @pl.when(cid == 1)
        def _():
            convert(embc1)

        plsc.subcore_barrier()

        def pool_all(embc):
            def start_gather(c, rows, sem):
                pltpu.async_copy(
                    embc.at[idx_v.at[pl.ds(c * (CH * SEQ), CH * SEQ)]],
                    rows, sem)

            def wait_gather(rows, sem):
                pltpu.make_async_copy(
                    embc.at[idx_v.at[pl.ds(0, CH * SEQ)]], rows, sem).wait()

            def start_out(c, acc, sem):
                pltpu.async_copy(acc, out_hbm.at[pl.ds(base + c * CH, CH)],
                                 sem)

            def wait_out(acc, sem):
                pltpu.make_async_copy(acc, out_hbm.at[pl.ds(base, CH)],
                                      sem).wait()

            def accumulate(rows, acc):
                def sbody(s, carry):
                    r = s * SEQ

                    def jbody(j, accs):
                        r0 = r + UN * j
                        new = list(accs)
                        for u in range(UN):
                            for g in range(G2):
                                w = rows[r0 + u, pl.ds(g * LANES, LANES)]
                                even = lax.bitcast_convert_type(
                                    w << 16, jnp.float32)
                                odd = lax.bitcast_convert_type(
                                    w & HIMASK, jnp.float32)
                                new[2 * g] = new[2 * g] + even
                                new[2 * g + 1] = new[2 * g + 1] + odd
                        return tuple(new)

                    accs = tuple(jnp.zeros((LANES,), jnp.float32)
                                 for _ in range(2 * G2))
                    accs = lax.fori_loop(0, SEQ // UN, jbody, accs)
                    for g in range(G2):
                        acc[s, pl.ds(32 * g, LANES)] = accs[2 * g]
                        acc[s, pl.ds(32 * g + LANES, LANES)] = accs[2 * g + 1]
                    return carry

                lax.fori_loop(0, CH, sbody, 0)

            start_gather(0, rows0, g0)

            def pair(k, carry):
                c0 = 2 * k
                wait_gather(rows0, g0)
                start_gather(c0 + 1, rows1, g1)

                @pl.when(k > 0)
                def _():
                    wait_out(acc0, o0)

                accumulate(rows0, acc0)
                start_out(c0, acc0, o0)
                wait_gather(rows1, g1)

                @pl.when(k < PAIRS - 1)
                def _():
                    start_gather(c0 + 2, rows0, g0)

                @pl.when(k > 0)
                def _():
                    wait_out(acc1, o1)

                accumulate(rows1, acc1)
                start_out(c0 + 1, acc1, o1)
                return carry

            lax.fori_loop(0, PAIRS, pair, 0)
            wait_out(acc0, o0)
            wait_out(acc1, o1)

        @pl.when(cid == 0)
        def _():
            pool_all(embc0)

        @pl.when(cid == 1)
        def _():
            pool_all(embc1)

    return pool_kernel(emb, idx_flat)[0]


def _tc_mlp_cos(pooled, W1, b1, W2, b2, W3, b3, batch):
    TB = 512
    grid = batch // TB

    def body(ea_ref, eb_ref, w1, bb1, w2, bb2, w3, bb3, out_ref):
        def mlp(x):
            h = jnp.tanh(
                lax.dot_general(x, w1[...], (((1,), (1,)), ((), ())),
                                preferred_element_type=jnp.float32) + bb1[...])
            h = jnp.tanh(
                lax.dot_general(h, w2[...], (((1,), (1,)), ((), ())),
                                preferred_element_type=jnp.float32) + bb2[...])
            h = jnp.tanh(
                lax.dot_general(h, w3[...], (((1,), (1,)), ((), ())),
                                preferred_element_type=jnp.float32) + bb3[...])
            return h

        fa = mlp(ea_ref[...])
        fb = mlp(eb_ref[...])
        eps = 1e-8
        na = jnp.maximum(jnp.sqrt(jnp.sum(fa * fa, axis=1, keepdims=True)), eps)
        nb = jnp.maximum(jnp.sqrt(jnp.sum(fb * fb, axis=1, keepdims=True)), eps)
        dot = jnp.sum(fa * fb, axis=1, keepdims=True)
        out_ref[...] = dot / (na * nb)

    out = pl.pallas_call(
        body,
        grid=(grid,),
        in_specs=[
            pl.BlockSpec((TB, EMBD), lambda i: (i, 0)),
            pl.BlockSpec((TB, EMBD), lambda i: (i + grid, 0)),
            pl.BlockSpec(W1.shape, lambda i: (0, 0)),
            pl.BlockSpec((1, 256), lambda i: (0, 0)),
            pl.BlockSpec(W2.shape, lambda i: (0, 0)),
            pl.BlockSpec((1, 128), lambda i: (0, 0)),
            pl.BlockSpec(W3.shape, lambda i: (0, 0)),
            pl.BlockSpec((1, 64), lambda i: (0, 0)),
        ],
        out_specs=pl.BlockSpec((TB, 1), lambda i: (i, 0)),
        out_shape=jax.ShapeDtypeStruct((batch, 1), jnp.float32),
    )(pooled, pooled, W1, b1.reshape(1, -1), W2, b2.reshape(1, -1),
      W3, b3.reshape(1, -1))
    return out.reshape(-1)


def kernel(a, b, emb, W1, b1, W2, b2, W3, b3):
    batch = a.shape[0]
    idx = jnp.concatenate([a, b], axis=0).astype(jnp.int32).reshape(-1)
    pooled = _sc_pool(emb, idx, 2 * batch)
    return _tc_mlp_cos(pooled, W1, b1, W2, b2, W3, b3, batch)


# speedup vs baseline: 1.1547x; 1.1547x over previous
import functools

import jax
import jax.numpy as jnp
from jax import lax
from jax.experimental import pallas as pl
from jax.experimental.pallas import tpu as pltpu
from jax.experimental.pallas import tpu_sc as plsc

EMBD = 128
SEQ = 50
LANES = 16
NCORES = 2
NSUB = 16
NW = NCORES * NSUB
G2 = EMBD // (2 * LANES)
WPR = EMBD // 2
RBIAS = jnp.int32(0x8000)
HIMASK = jnp.int32(-65536)


def _sc_pool(emb, idx_flat, n_samples):
    vocab = emb.shape[0]
    per_w = n_samples // NW
    CH = 8
    PAIRS = per_w // (2 * CH)
    UN = 5
    CROWS = 66
    rows_per_tile = vocab // NSUB
    tail = vocab - rows_per_tile * NSUB
    csteps = rows_per_tile // CROWS
    assert rows_per_tile % CROWS == 0
    mesh = plsc.VectorSubcoreMesh(core_axis_name="c", subcore_axis_name="s")

    @functools.partial(
        pl.kernel,
        mesh=mesh,
        out_type=(
            jax.ShapeDtypeStruct((n_samples, EMBD), jnp.float32),
            jax.ShapeDtypeStruct((vocab, WPR), jnp.int32),
            jax.ShapeDtypeStruct((vocab, WPR), jnp.int32),
        ),
        scratch_types=[
            pltpu.VMEM((CROWS, EMBD), jnp.float32),
            pltpu.VMEM((CROWS, EMBD), jnp.float32),
            pltpu.VMEM((CROWS, WPR), jnp.int32),
            pltpu.VMEM((CROWS, WPR), jnp.int32),
            pltpu.VMEM((per_w * SEQ,), jnp.int32),
            pltpu.VMEM((CH * SEQ, WPR), jnp.int32),
            pltpu.VMEM((CH * SEQ, WPR), jnp.int32),
            pltpu.VMEM((CH, EMBD), jnp.float32),
            pltpu.VMEM((CH, EMBD), jnp.float32),
            pltpu.SemaphoreType.DMA,
            pltpu.SemaphoreType.DMA,
            pltpu.SemaphoreType.DMA,
            pltpu.SemaphoreType.DMA,
            pltpu.SemaphoreType.DMA,
            pltpu.SemaphoreType.DMA,
            pltpu.SemaphoreType.DMA,
            pltpu.SemaphoreType.DMA,
        ],
        compiler_params=pltpu.CompilerParams(use_tc_tiling_on_sc=False),
    )
    def pool_kernel(emb_hbm, idx_hbm, out_hbm, embc0, embc1,
                    fin0, fin1, fout0, fout1, idx_v, rows0, rows1,
                    acc0, acc1, g0, g1, o0, o1, ci0, ci1, co0, co1):
        cid = lax.axis_index("c")
        sid = lax.axis_index("s")
        wid = sid * NCORES + cid
        base = wid * per_w
        pltpu.sync_copy(idx_hbm.at[pl.ds(base * SEQ, per_w * SEQ)], idx_v)

        fins = (fin0, fin1)
        fouts = (fout0, fout1)
        cins = (ci0, ci1)
        couts = (co0, co1)
        conv_r0 = sid * rows_per_tile

        def conv_compute(fin, fout):
            def rbody(r, carry):
                for g in range(G2):
                    e = lax.bitcast_convert_type(
                        fin[r, pl.ds(32 * g, LANES)], jnp.int32)
                    o = lax.bitcast_convert_type(
                        fin[r, pl.ds(32 * g + LANES, LANES)], jnp.int32)
                    lo = lax.shift_right_logical(e + RBIAS, 16)
                    hi = (o + RBIAS) & HIMASK
                    fout[r, pl.ds(LANES * g, LANES)] = hi | lo
                return carry

            lax.fori_loop(0, CROWS, rbody, 0)

        def conv_all(embc):
            for k in range(min(2, csteps)):
                pltpu.async_copy(
                    emb_hbm.at[pl.ds(conv_r0 + k * CROWS, CROWS)],
                    fins[k % 2], cins[k % 2])
            for k in range(csteps):
                b = k % 2
                pltpu.make_async_copy(
                    emb_hbm.at[pl.ds(conv_r0, CROWS)], fins[b],
                    cins[b]).wait()
                if k >= 2:
                    pltpu.make_async_copy(
                        fouts[b], embc.at[pl.ds(conv_r0, CROWS)],
                        couts[b]).wait()
                conv_compute(fins[b], fouts[b])
                pltpu.async_copy(
                    fouts[b], embc.at[pl.ds(conv_r0 + k * CROWS, CROWS)],
                    couts[b])
                if k + 2 < csteps:
                    pltpu.async_copy(
                        emb_hbm.at[pl.ds(conv_r0 + (k + 2) * CROWS, CROWS)],
                        fins[b], cins[b])
            for k in range(max(0, csteps - 2), csteps):
                b = k % 2
                pltpu.make_async_copy(
                    fouts[b], embc.at[pl.ds(conv_r0, CROWS)],
                    couts[b]).wait()

            @pl.when(sid == 0)
            def _():
                pltpu.sync_copy(emb_hbm.at[pl.ds(vocab - tail, tail)],
                                fin0.at[pl.ds(0, tail)])
                conv_compute(fin0, fout0)
                pltpu.sync_copy(fout0.at[pl.ds(0, tail)],
                                embc.at[pl.ds(vocab - tail, tail)])

        @pl.when(cid == 0)
        def _():
            conv_all(embc0)

        @pl.when(cid == 1)
        def _():
            conv_all(embc1)

        plsc.subcore_barrier()

        def pool_all(embc):
            def start_gather(c, rows, sem):
                pltpu.async_copy(
                    embc.at[idx_v.at[pl.ds(c * (CH * SEQ), CH * SEQ)]],
                    rows, sem)

            def wait_gather(rows, sem):
                pltpu.make_async_copy(
                    embc.at[idx_v.at[pl.ds(0, CH * SEQ)]], rows, sem).wait()

            def start_out(c, acc, sem):
                pltpu.async_copy(acc, out_hbm.at[pl.ds(base + c * CH, CH)],
                                 sem)

            def wait_out(acc, sem):
                pltpu.make_async_copy(acc, out_hbm.at[pl.ds(base, CH)],
                                      sem).wait()

            def accumulate(rows, acc):
                def sbody(s, carry):
                    r = s * SEQ

                    def jbody(j, accs):
                        r0 = r + UN * j
                        new = list(accs)
                        for u in range(UN):
                            for g in range(G2):
                                w = rows[r0 + u, pl.ds(g * LANES, LANES)]
                                even = lax.bitcast_convert_type(
                                    w << 16, jnp.float32)
                                odd = lax.bitcast_convert_type(
                                    w & HIMASK, jnp.float32)
                                new[2 * g] = new[2 * g] + even
                                new[2 * g + 1] = new[2 * g + 1] + odd
                        return tuple(new)

                    accs = tuple(jnp.zeros((LANES,), jnp.float32)
                                 for _ in range(2 * G2))
                    accs = lax.fori_loop(0, SEQ // UN, jbody, accs)
                    for g in range(G2):
                        acc[s, pl.ds(32 * g, LANES)] = accs[2 * g]
                        acc[s, pl.ds(32 * g + LANES, LANES)] = accs[2 * g + 1]
                    return carry

                lax.fori_loop(0, CH, sbody, 0)

            start_gather(0, rows0, g0)

            def pair(k, carry):
                c0 = 2 * k
                wait_gather(rows0, g0)
                start_gather(c0 + 1, rows1, g1)

                @pl.when(k > 0)
                def _():
                    wait_out(acc0, o0)

                accumulate(rows0, acc0)
                start_out(c0, acc0, o0)
                wait_gather(rows1, g1)

                @pl.when(k < PAIRS - 1)
                def _():
                    start_gather(c0 + 2, rows0, g0)

                @pl.when(k > 0)
                def _():
                    wait_out(acc1, o1)

                accumulate(rows1, acc1)
                start_out(c0 + 1, acc1, o1)
                return carry

            lax.fori_loop(0, PAIRS, pair, 0)
            wait_out(acc0, o0)
            wait_out(acc1, o1)

        @pl.when(cid == 0)
        def _():
            pool_all(embc0)

        @pl.when(cid == 1)
        def _():
            pool_all(embc1)

    return pool_kernel(emb, idx_flat)[0]


def _tc_mlp_cos(pooled, W1, b1, W2, b2, W3, b3, batch):
    TB = 512
    grid = batch // TB

    def body(ea_ref, eb_ref, w1, bb1, w2, bb2, w3, bb3, out_ref):
        def mlp(x):
            h = jnp.tanh(
                lax.dot_general(x, w1[...], (((1,), (1,)), ((), ())),
                                preferred_element_type=jnp.float32) + bb1[...])
            h = jnp.tanh(
                lax.dot_general(h, w2[...], (((1,), (1,)), ((), ())),
                                preferred_element_type=jnp.float32) + bb2[...])
            h = jnp.tanh(
                lax.dot_general(h, w3[...], (((1,), (1,)), ((), ())),
                                preferred_element_type=jnp.float32) + bb3[...])
            return h

        fa = mlp(ea_ref[...])
        fb = mlp(eb_ref[...])
        eps = 1e-8
        na = jnp.maximum(jnp.sqrt(jnp.sum(fa * fa, axis=1, keepdims=True)), eps)
        nb = jnp.maximum(jnp.sqrt(jnp.sum(fb * fb, axis=1, keepdims=True)), eps)
        dot = jnp.sum(fa * fb, axis=1, keepdims=True)
        out_ref[...] = dot / (na * nb)

    out = pl.pallas_call(
        body,
        grid=(grid,),
        in_specs=[
            pl.BlockSpec((TB, EMBD), lambda i: (i, 0)),
            pl.BlockSpec((TB, EMBD), lambda i: (i + grid, 0)),
            pl.BlockSpec(W1.shape, lambda i: (0, 0)),
            pl.BlockSpec((1, 256), lambda i: (0, 0)),
            pl.BlockSpec(W2.shape, lambda i: (0, 0)),
            pl.BlockSpec((1, 128), lambda i: (0, 0)),
            pl.BlockSpec(W3.shape, lambda i: (0, 0)),
            pl.BlockSpec((1, 64), lambda i: (0, 0)),
        ],
        out_specs=pl.BlockSpec((TB, 1), lambda i: (i, 0)),
        out_shape=jax.ShapeDtypeStruct((batch, 1), jnp.float32),
    )(pooled, pooled, W1, b1.reshape(1, -1), W2, b2.reshape(1, -1),
      W3, b3.reshape(1, -1))
    return out.reshape(-1)


def kernel(a, b, emb, W1, b1, W2, b2, W3, b3):
    batch = a.shape[0]
    idx = jnp.concatenate([a, b], axis=0).astype(jnp.int32).reshape(-1)
    pooled = _sc_pool(emb, idx, 2 * batch)
    return _tc_mlp_cos(pooled, W1, b1, W2, b2, W3, b3, batch)


# speedup vs baseline: 1.1738x; 1.0166x over previous
import functools

import jax
import jax.numpy as jnp
from jax import lax
from jax.experimental import pallas as pl
from jax.experimental.pallas import tpu as pltpu
from jax.experimental.pallas import tpu_sc as plsc

EMBD = 128
SEQ = 50
LANES = 16
NCORES = 2
NSUB = 16
NW = NCORES * NSUB
G2 = EMBD // (2 * LANES)
WPR = EMBD // 2
RBIAS = jnp.int32(0x8000)
HIMASK = jnp.int32(-65536)


def _sc_pool(emb, idx_flat, n_samples):
    vocab = emb.shape[0]
    per_w = n_samples // NW
    CH = 8
    PAIRS = per_w // (2 * CH)
    UN = 5
    CROWS = 66
    rows_per_tile = vocab // NSUB
    tail = vocab - rows_per_tile * NSUB
    csteps = rows_per_tile // CROWS
    assert rows_per_tile % CROWS == 0
    mesh = plsc.VectorSubcoreMesh(core_axis_name="c", subcore_axis_name="s")

    @functools.partial(
        pl.kernel,
        mesh=mesh,
        out_type=(
            jax.ShapeDtypeStruct((n_samples, EMBD), jnp.float32),
            jax.ShapeDtypeStruct((vocab, WPR), jnp.int32),
            jax.ShapeDtypeStruct((vocab, WPR), jnp.int32),
        ),
        scratch_types=[
            pltpu.VMEM((CROWS, EMBD), jnp.float32),
            pltpu.VMEM((CROWS, EMBD), jnp.float32),
            pltpu.VMEM((CROWS, WPR), jnp.int32),
            pltpu.VMEM((CROWS, WPR), jnp.int32),
            pltpu.VMEM((per_w * SEQ,), jnp.int32),
            pltpu.VMEM((CH * SEQ, WPR), jnp.int32),
            pltpu.VMEM((CH * SEQ, WPR), jnp.int32),
            pltpu.VMEM((CH, EMBD), jnp.float32),
            pltpu.VMEM((CH, EMBD), jnp.float32),
            pltpu.SemaphoreType.DMA,
            pltpu.SemaphoreType.DMA,
            pltpu.SemaphoreType.DMA,
            pltpu.SemaphoreType.DMA,
            pltpu.SemaphoreType.DMA,
            pltpu.SemaphoreType.DMA,
            pltpu.SemaphoreType.DMA,
            pltpu.SemaphoreType.DMA,
        ],
        compiler_params=pltpu.CompilerParams(use_tc_tiling_on_sc=False),
    )
    def pool_kernel(emb_hbm, idxa_hbm, idxb_hbm, out_hbm, embc0, embc1,
                    fin0, fin1, fout0, fout1, idx_v, rows0, rows1,
                    acc0, acc1, g0, g1, o0, o1, ci0, ci1, co0, co1):
        cid = lax.axis_index("c")
        sid = lax.axis_index("s")
        base = cid * (NSUB * per_w) + sid * per_w

        @pl.when(cid == 0)
        def _():
            pltpu.sync_copy(
                idxa_hbm.at[pl.ds(sid * per_w * SEQ, per_w * SEQ)], idx_v)

        @pl.when(cid == 1)
        def _():
            pltpu.sync_copy(
                idxb_hbm.at[pl.ds(sid * per_w * SEQ, per_w * SEQ)], idx_v)

        fins = (fin0, fin1)
        fouts = (fout0, fout1)
        cins = (ci0, ci1)
        couts = (co0, co1)
        conv_r0 = sid * rows_per_tile

        def conv_compute(fin, fout):
            def rbody(rr, carry):
                for u in range(2):
                    r = 2 * rr + u
                    for g in range(G2):
                        e = lax.bitcast_convert_type(
                            fin[r, pl.ds(32 * g, LANES)], jnp.int32)
                        o = lax.bitcast_convert_type(
                            fin[r, pl.ds(32 * g + LANES, LANES)], jnp.int32)
                        lo = lax.shift_right_logical(e + RBIAS, 16)
                        hi = (o + RBIAS) & HIMASK
                        fout[r, pl.ds(LANES * g, LANES)] = hi | lo
                return carry

            lax.fori_loop(0, CROWS // 2, rbody, 0)

        def conv_all(embc):
            for k in range(min(2, csteps)):
                pltpu.async_copy(
                    emb_hbm.at[pl.ds(conv_r0 + k * CROWS, CROWS)],
                    fins[k % 2], cins[k % 2])
            for k in range(csteps):
                b = k % 2
                pltpu.make_async_copy(
                    emb_hbm.at[pl.ds(conv_r0, CROWS)], fins[b],
                    cins[b]).wait()
                if k >= 2:
                    pltpu.make_async_copy(
                        fouts[b], embc.at[pl.ds(conv_r0, CROWS)],
                        couts[b]).wait()
                conv_compute(fins[b], fouts[b])
                pltpu.async_copy(
                    fouts[b], embc.at[pl.ds(conv_r0 + k * CROWS, CROWS)],
                    couts[b])
                if k + 2 < csteps:
                    pltpu.async_copy(
                        emb_hbm.at[pl.ds(conv_r0 + (k + 2) * CROWS, CROWS)],
                        fins[b], cins[b])
            for k in range(max(0, csteps - 2), csteps):
                b = k % 2
                pltpu.make_async_copy(
                    fouts[b], embc.at[pl.ds(conv_r0, CROWS)],
                    couts[b]).wait()

            @pl.when(sid == 0)
            def _():
                pltpu.sync_copy(emb_hbm.at[pl.ds(vocab - tail, tail)],
                                fin0.at[pl.ds(0, tail)])
                conv_compute(fin0, fout0)
                pltpu.sync_copy(fout0.at[pl.ds(0, tail)],
                                embc.at[pl.ds(vocab - tail, tail)])

        @pl.when(cid == 0)
        def _():
            conv_all(embc0)

        @pl.when(cid == 1)
        def _():
            conv_all(embc1)

        plsc.subcore_barrier()

        def pool_all(embc):
            def start_gather(c, rows, sem):
                pltpu.async_copy(
                    embc.at[idx_v.at[pl.ds(c * (CH * SEQ), CH * SEQ)]],
                    rows, sem)

            def wait_gather(rows, sem):
                pltpu.make_async_copy(
                    embc.at[idx_v.at[pl.ds(0, CH * SEQ)]], rows, sem).wait()

            def start_out(c, acc, sem):
                pltpu.async_copy(acc, out_hbm.at[pl.ds(base + c * CH, CH)],
                                 sem)

            def wait_out(acc, sem):
                pltpu.make_async_copy(acc, out_hbm.at[pl.ds(base, CH)],
                                      sem).wait()

            def accumulate(rows, acc):
                def sbody(s, carry):
                    r = s * SEQ

                    def jbody(j, accs):
                        r0 = r + UN * j
                        new = list(accs)
                        for u in range(UN):
                            for g in range(G2):
                                w = rows[r0 + u, pl.ds(g * LANES, LANES)]
                                even = lax.bitcast_convert_type(
                                    w << 16, jnp.float32)
                                odd = lax.bitcast_convert_type(
                                    w & HIMASK, jnp.float32)
                                new[2 * g] = new[2 * g] + even
                                new[2 * g + 1] = new[2 * g + 1] + odd
                        return tuple(new)

                    accs = tuple(jnp.zeros((LANES,), jnp.float32)
                                 for _ in range(2 * G2))
                    accs = lax.fori_loop(0, SEQ // UN, jbody, accs)
                    for g in range(G2):
                        acc[s, pl.ds(32 * g, LANES)] = accs[2 * g]
                        acc[s, pl.ds(32 * g + LANES, LANES)] = accs[2 * g + 1]
                    return carry

                lax.fori_loop(0, CH, sbody, 0)

            start_gather(0, rows0, g0)

            def pair(k, carry):
                c0 = 2 * k
                wait_gather(rows0, g0)
                start_gather(c0 + 1, rows1, g1)

                @pl.when(k > 0)
                def _():
                    wait_out(acc0, o0)

                accumulate(rows0, acc0)
                start_out(c0, acc0, o0)
                wait_gather(rows1, g1)

                @pl.when(k < PAIRS - 1)
                def _():
                    start_gather(c0 + 2, rows0, g0)

                @pl.when(k > 0)
                def _():
                    wait_out(acc1, o1)

                accumulate(rows1, acc1)
                start_out(c0 + 1, acc1, o1)
                return carry

            lax.fori_loop(0, PAIRS, pair, 0)
            wait_out(acc0, o0)
            wait_out(acc1, o1)

        @pl.when(cid == 0)
        def _():
            pool_all(embc0)

        @pl.when(cid == 1)
        def _():
            pool_all(embc1)

    return pool_kernel(emb, idx_flat[0], idx_flat[1])[0]


def _tc_mlp_cos(pooled, W1, b1, W2, b2, W3, b3, batch):
    TB = 512
    grid = batch // TB

    def body(ea_ref, eb_ref, w1, bb1, w2, bb2, w3, bb3, out_ref):
        def mlp(x):
            h = jnp.tanh(
                lax.dot_general(x, w1[...], (((1,), (1,)), ((), ())),
                                preferred_element_type=jnp.float32) + bb1[...])
            h = jnp.tanh(
                lax.dot_general(h, w2[...], (((1,), (1,)), ((), ())),
                                preferred_element_type=jnp.float32) + bb2[...])
            h = jnp.tanh(
                lax.dot_general(h, w3[...], (((1,), (1,)), ((), ())),
                                preferred_element_type=jnp.float32) + bb3[...])
            return h

        fa = mlp(ea_ref[...])
        fb = mlp(eb_ref[...])
        eps = 1e-8
        na = jnp.maximum(jnp.sqrt(jnp.sum(fa * fa, axis=1, keepdims=True)), eps)
        nb = jnp.maximum(jnp.sqrt(jnp.sum(fb * fb, axis=1, keepdims=True)), eps)
        dot = jnp.sum(fa * fb, axis=1, keepdims=True)
        out_ref[...] = dot / (na * nb)

    out = pl.pallas_call(
        body,
        grid=(grid,),
        in_specs=[
            pl.BlockSpec((TB, EMBD), lambda i: (i, 0)),
            pl.BlockSpec((TB, EMBD), lambda i: (i + grid, 0)),
            pl.BlockSpec(W1.shape, lambda i: (0, 0)),
            pl.BlockSpec((1, 256), lambda i: (0, 0)),
            pl.BlockSpec(W2.shape, lambda i: (0, 0)),
            pl.BlockSpec((1, 128), lambda i: (0, 0)),
            pl.BlockSpec(W3.shape, lambda i: (0, 0)),
            pl.BlockSpec((1, 64), lambda i: (0, 0)),
        ],
        out_specs=pl.BlockSpec((TB, 1), lambda i: (i, 0)),
        out_shape=jax.ShapeDtypeStruct((batch, 1), jnp.float32),
    )(pooled, pooled, W1, b1.reshape(1, -1), W2, b2.reshape(1, -1),
      W3, b3.reshape(1, -1))
    return out.reshape(-1)


def kernel(a, b, emb, W1, b1, W2, b2, W3, b3):
    batch = a.shape[0]
    af = a.astype(jnp.int32).reshape(-1)
    bf = b.astype(jnp.int32).reshape(-1)
    pooled = _sc_pool(emb, (af, bf), 2 * batch)
    return _tc_mlp_cos(pooled, W1, b1, W2, b2, W3, b3, batch)


# speedup vs baseline: 1.2653x; 1.0779x over previous
import functools

import jax
import jax.numpy as jnp
from jax import lax
from jax.experimental import pallas as pl
from jax.experimental.pallas import tpu as pltpu
from jax.experimental.pallas import tpu_sc as plsc

EMBD = 128
SEQ = 50
LANES = 16
NCORES = 2
NSUB = 16
NW = NCORES * NSUB
G2 = EMBD // (2 * LANES)
WPR = EMBD // 2
RBIAS = 0x8000
HIMASK = -65536


def _sc_pool(emb, idx_flat, n_samples):
    vocab = emb.shape[0]
    per_w = n_samples // NW
    CH = 8
    PAIRS = per_w // (2 * CH)
    UN = 5
    mesh = plsc.VectorSubcoreMesh(core_axis_name="c", subcore_axis_name="s")

    @functools.partial(
        pl.kernel,
        mesh=mesh,
        out_type=jax.ShapeDtypeStruct((n_samples, EMBD), jnp.float32),
        scratch_types=[
            pltpu.VMEM((per_w * SEQ,), jnp.int32),
            pltpu.VMEM((CH * SEQ, WPR), jnp.int32),
            pltpu.VMEM((CH * SEQ, WPR), jnp.int32),
            pltpu.VMEM((CH, EMBD), jnp.float32),
            pltpu.VMEM((CH, EMBD), jnp.float32),
            pltpu.SemaphoreType.DMA,
            pltpu.SemaphoreType.DMA,
            pltpu.SemaphoreType.DMA,
            pltpu.SemaphoreType.DMA,
        ],
        compiler_params=pltpu.CompilerParams(use_tc_tiling_on_sc=False),
    )
    def pool_kernel(embp_hbm, idxa_hbm, idxb_hbm, out_hbm,
                    idx_v, rows0, rows1,
                    acc0, acc1, g0, g1, o0, o1):
        cid = lax.axis_index("c")
        sid = lax.axis_index("s")
        base = cid * (NSUB * per_w) + sid * per_w

        @pl.when(cid == 0)
        def _():
            pltpu.sync_copy(
                idxa_hbm.at[pl.ds(sid * per_w * SEQ, per_w * SEQ)], idx_v)

        @pl.when(cid == 1)
        def _():
            pltpu.sync_copy(
                idxb_hbm.at[pl.ds(sid * per_w * SEQ, per_w * SEQ)], idx_v)


        def pool_all(embc):
            def start_gather(c, rows, sem):
                pltpu.async_copy(
                    embc.at[idx_v.at[pl.ds(c * (CH * SEQ), CH * SEQ)]],
                    rows, sem)

            def wait_gather(rows, sem):
                pltpu.make_async_copy(
                    embc.at[idx_v.at[pl.ds(0, CH * SEQ)]], rows, sem).wait()

            def start_out(c, acc, sem):
                pltpu.async_copy(acc, out_hbm.at[pl.ds(base + c * CH, CH)],
                                 sem)

            def wait_out(acc, sem):
                pltpu.make_async_copy(acc, out_hbm.at[pl.ds(base, CH)],
                                      sem).wait()

            def accumulate(rows, acc):
                def sbody(s, carry):
                    r = s * SEQ

                    def jbody(j, accs):
                        r0 = r + UN * j
                        new = list(accs)
                        for u in range(UN):
                            for g in range(G2):
                                w = rows[r0 + u, pl.ds(g * LANES, LANES)]
                                even = lax.bitcast_convert_type(
                                    w << 16, jnp.float32)
                                odd = lax.bitcast_convert_type(
                                    w & HIMASK, jnp.float32)
                                new[2 * g] = new[2 * g] + even
                                new[2 * g + 1] = new[2 * g + 1] + odd
                        return tuple(new)

                    accs = tuple(jnp.zeros((LANES,), jnp.float32)
                                 for _ in range(2 * G2))
                    accs = lax.fori_loop(0, SEQ // UN, jbody, accs)
                    for g in range(G2):
                        acc[s, pl.ds(32 * g, LANES)] = accs[2 * g]
                        acc[s, pl.ds(32 * g + LANES, LANES)] = accs[2 * g + 1]
                    return carry

                lax.fori_loop(0, CH, sbody, 0)

            start_gather(0, rows0, g0)

            def pair(k, carry):
                c0 = 2 * k
                wait_gather(rows0, g0)
                start_gather(c0 + 1, rows1, g1)

                @pl.when(k > 0)
                def _():
                    wait_out(acc0, o0)

                accumulate(rows0, acc0)
                start_out(c0, acc0, o0)
                wait_gather(rows1, g1)

                @pl.when(k < PAIRS - 1)
                def _():
                    start_gather(c0 + 2, rows0, g0)

                @pl.when(k > 0)
                def _():
                    wait_out(acc1, o1)

                accumulate(rows1, acc1)
                start_out(c0 + 1, acc1, o1)
                return carry

            lax.fori_loop(0, PAIRS, pair, 0)
            wait_out(acc0, o0)
            wait_out(acc1, o1)

        pool_all(embp_hbm)

    return pool_kernel(emb, idx_flat[0], idx_flat[1])


def _tc_pack(emb2):
    half, width = emb2.shape

    def body(x_ref, out_ref):
        r = lax.bitcast_convert_type(x_ref[...], jnp.int32) + RBIAS
        words = []
        for h in range(2):
            for g in range(G2):
                c = 128 * h + 32 * g
                lo = lax.shift_right_logical(r[:, c:c + LANES], 16)
                hi = r[:, c + LANES:c + 2 * LANES] & HIMASK
                words.append(hi | lo)
        out_ref[...] = jnp.concatenate(words, axis=1)

    return pl.pallas_call(
        body,
        in_specs=[pl.BlockSpec((half, width), lambda: (0, 0))],
        out_specs=pl.BlockSpec((half, EMBD), lambda: (0, 0)),
        out_shape=jax.ShapeDtypeStruct((half, EMBD), jnp.int32),
    )(emb2)


def _tc_mlp_cos(pooled, W1, b1, W2, b2, W3, b3, batch):
    TB = 512
    grid = batch // TB

    def body(ea_ref, eb_ref, w1, bb1, w2, bb2, w3, bb3, out_ref):
        def mlp(x):
            h = jnp.tanh(
                lax.dot_general(x, w1[...], (((1,), (1,)), ((), ())),
                                preferred_element_type=jnp.float32) + bb1[...])
            h = jnp.tanh(
                lax.dot_general(h, w2[...], (((1,), (1,)), ((), ())),
                                preferred_element_type=jnp.float32) + bb2[...])
            h = jnp.tanh(
                lax.dot_general(h, w3[...], (((1,), (1,)), ((), ())),
                                preferred_element_type=jnp.float32) + bb3[...])
            return h

        fa = mlp(ea_ref[...])
        fb = mlp(eb_ref[...])
        eps = 1e-8
        na = jnp.maximum(jnp.sqrt(jnp.sum(fa * fa, axis=1, keepdims=True)), eps)
        nb = jnp.maximum(jnp.sqrt(jnp.sum(fb * fb, axis=1, keepdims=True)), eps)
        dot = jnp.sum(fa * fb, axis=1, keepdims=True)
        out_ref[...] = dot / (na * nb)

    out = pl.pallas_call(
        body,
        grid=(grid,),
        in_specs=[
            pl.BlockSpec((TB, EMBD), lambda i: (i, 0)),
            pl.BlockSpec((TB, EMBD), lambda i: (i + grid, 0)),
            pl.BlockSpec(W1.shape, lambda i: (0, 0)),
            pl.BlockSpec((1, 256), lambda i: (0, 0)),
            pl.BlockSpec(W2.shape, lambda i: (0, 0)),
            pl.BlockSpec((1, 128), lambda i: (0, 0)),
            pl.BlockSpec(W3.shape, lambda i: (0, 0)),
            pl.BlockSpec((1, 64), lambda i: (0, 0)),
        ],
        out_specs=pl.BlockSpec((TB, 1), lambda i: (i, 0)),
        out_shape=jax.ShapeDtypeStruct((batch, 1), jnp.float32),
    )(pooled, pooled, W1, b1.reshape(1, -1), W2, b2.reshape(1, -1),
      W3, b3.reshape(1, -1))
    return out.reshape(-1)


def kernel(a, b, emb, W1, b1, W2, b2, W3, b3):
    batch = a.shape[0]
    af = a.astype(jnp.int32).reshape(-1)
    bf = b.astype(jnp.int32).reshape(-1)
    vocab = emb.shape[0]
    packed = _tc_pack(emb.reshape(vocab // 2, 2 * EMBD))
    embp = packed.reshape(vocab, WPR)
    pooled = _sc_pool(embp, (af, bf), 2 * batch)
    return _tc_mlp_cos(pooled, W1, b1, W2, b2, W3, b3, batch)


# speedup vs baseline: 1.3879x; 1.0969x over previous
import functools

import jax
import jax.numpy as jnp
from jax import lax
from jax.experimental import pallas as pl
from jax.experimental.pallas import tpu as pltpu
from jax.experimental.pallas import tpu_sc as plsc

EMBD = 128
SEQ = 50
LANES = 16
NCORES = 2
NSUB = 16
NW = NCORES * NSUB
G2 = EMBD // (2 * LANES)
WPR = EMBD // 2
RBIAS = 0x8000
HIMASK = -65536


def _sc_pool(emb, idx_flat, n_samples):
    vocab = emb.shape[0]
    per_w = n_samples // NW
    CH = 16
    PAIRS = per_w // (2 * CH)
    UN = 5
    mesh = plsc.VectorSubcoreMesh(core_axis_name="c", subcore_axis_name="s")

    @functools.partial(
        pl.kernel,
        mesh=mesh,
        out_type=jax.ShapeDtypeStruct((n_samples, EMBD), jnp.float32),
        scratch_types=[
            pltpu.VMEM((per_w * SEQ,), jnp.int32),
            pltpu.VMEM((CH * SEQ, WPR), jnp.int32),
            pltpu.VMEM((CH * SEQ, WPR), jnp.int32),
            pltpu.VMEM((CH, EMBD), jnp.float32),
            pltpu.VMEM((CH, EMBD), jnp.float32),
            pltpu.SemaphoreType.DMA,
            pltpu.SemaphoreType.DMA,
            pltpu.SemaphoreType.DMA,
            pltpu.SemaphoreType.DMA,
        ],
        compiler_params=pltpu.CompilerParams(use_tc_tiling_on_sc=False),
    )
    def pool_kernel(embp_hbm, idxa_hbm, idxb_hbm, out_hbm,
                    idx_v, rows0, rows1,
                    acc0, acc1, g0, g1, o0, o1):
        cid = lax.axis_index("c")
        sid = lax.axis_index("s")
        base = cid * (NSUB * per_w) + sid * per_w

        @pl.when(cid == 0)
        def _():
            pltpu.sync_copy(
                idxa_hbm.at[pl.ds(sid * per_w * SEQ, per_w * SEQ)], idx_v)

        @pl.when(cid == 1)
        def _():
            pltpu.sync_copy(
                idxb_hbm.at[pl.ds(sid * per_w * SEQ, per_w * SEQ)], idx_v)


        def pool_all(embc):
            def start_gather(c, rows, sem):
                pltpu.async_copy(
                    embc.at[idx_v.at[pl.ds(c * (CH * SEQ), CH * SEQ)]],
                    rows, sem)

            def wait_gather(rows, sem):
                pltpu.make_async_copy(
                    embc.at[idx_v.at[pl.ds(0, CH * SEQ)]], rows, sem).wait()

            def start_out(c, acc, sem):
                pltpu.async_copy(acc, out_hbm.at[pl.ds(base + c * CH, CH)],
                                 sem)

            def wait_out(acc, sem):
                pltpu.make_async_copy(acc, out_hbm.at[pl.ds(base, CH)],
                                      sem).wait()

            def accumulate(rows, acc):
                def sbody(s, carry):
                    r = s * SEQ

                    def jbody(j, accs):
                        r0 = r + UN * j
                        new = list(accs)
                        for u in range(UN):
                            for g in range(G2):
                                w = rows[r0 + u, pl.ds(g * LANES, LANES)]
                                even = lax.bitcast_convert_type(
                                    w << 16, jnp.float32)
                                odd = lax.bitcast_convert_type(
                                    w & HIMASK, jnp.float32)
                                new[2 * g] = new[2 * g] + even
                                new[2 * g + 1] = new[2 * g + 1] + odd
                        return tuple(new)

                    accs = tuple(jnp.zeros((LANES,), jnp.float32)
                                 for _ in range(2 * G2))
                    accs = lax.fori_loop(0, SEQ // UN, jbody, accs)
                    for g in range(G2):
                        acc[s, pl.ds(32 * g, LANES)] = accs[2 * g]
                        acc[s, pl.ds(32 * g + LANES, LANES)] = accs[2 * g + 1]
                    return carry

                lax.fori_loop(0, CH, sbody, 0)

            start_gather(0, rows0, g0)

            def pair(k, carry):
                c0 = 2 * k
                wait_gather(rows0, g0)
                start_gather(c0 + 1, rows1, g1)

                @pl.when(k > 0)
                def _():
                    wait_out(acc0, o0)

                accumulate(rows0, acc0)
                start_out(c0, acc0, o0)
                wait_gather(rows1, g1)

                @pl.when(k < PAIRS - 1)
                def _():
                    start_gather(c0 + 2, rows0, g0)

                @pl.when(k > 0)
                def _():
                    wait_out(acc1, o1)

                accumulate(rows1, acc1)
                start_out(c0 + 1, acc1, o1)
                return carry

            lax.fori_loop(0, PAIRS, pair, 0)
            wait_out(acc0, o0)
            wait_out(acc1, o1)

        pool_all(embp_hbm)

    return pool_kernel(emb, idx_flat[0], idx_flat[1])


def _tc_pack(emb2):
    half, width = emb2.shape

    def body(x_ref, out_ref):
        r = lax.bitcast_convert_type(x_ref[...], jnp.int32) + RBIAS
        words = []
        for h in range(2):
            for g in range(G2):
                c = 128 * h + 32 * g
                lo = lax.shift_right_logical(r[:, c:c + LANES], 16)
                hi = r[:, c + LANES:c + 2 * LANES] & HIMASK
                words.append(hi | lo)
        out_ref[...] = jnp.concatenate(words, axis=1)

    blk = 2648
    return pl.pallas_call(
        body,
        grid=((half + blk - 1) // blk,),
        in_specs=[pl.BlockSpec((blk, width), lambda i: (i, 0))],
        out_specs=pl.BlockSpec((blk, EMBD), lambda i: (i, 0)),
        out_shape=jax.ShapeDtypeStruct((half, EMBD), jnp.int32),
    )(emb2)


def _tc_mlp_cos(pooled, W1, b1, W2, b2, W3, b3, batch):
    TB = 512
    grid = batch // TB

    def body(ea_ref, eb_ref, w1, bb1, w2, bb2, w3, bb3, out_ref):
        def mlp(x):
            h = jnp.tanh(
                lax.dot_general(x, w1[...], (((1,), (1,)), ((), ())),
                                preferred_element_type=jnp.float32) + bb1[...])
            h = jnp.tanh(
                lax.dot_general(h, w2[...], (((1,), (1,)), ((), ())),
                                preferred_element_type=jnp.float32) + bb2[...])
            h = jnp.tanh(
                lax.dot_general(h, w3[...], (((1,), (1,)), ((), ())),
                                preferred_element_type=jnp.float32) + bb3[...])
            return h

        fa = mlp(ea_ref[...])
        fb = mlp(eb_ref[...])
        eps = 1e-8
        na = jnp.maximum(jnp.sqrt(jnp.sum(fa * fa, axis=1, keepdims=True)), eps)
        nb = jnp.maximum(jnp.sqrt(jnp.sum(fb * fb, axis=1, keepdims=True)), eps)
        dot = jnp.sum(fa * fb, axis=1, keepdims=True)
        out_ref[...] = dot / (na * nb)

    out = pl.pallas_call(
        body,
        grid=(grid,),
        in_specs=[
            pl.BlockSpec((TB, EMBD), lambda i: (i, 0)),
            pl.BlockSpec((TB, EMBD), lambda i: (i + grid, 0)),
            pl.BlockSpec(W1.shape, lambda i: (0, 0)),
            pl.BlockSpec((1, 256), lambda i: (0, 0)),
            pl.BlockSpec(W2.shape, lambda i: (0, 0)),
            pl.BlockSpec((1, 128), lambda i: (0, 0)),
            pl.BlockSpec(W3.shape, lambda i: (0, 0)),
            pl.BlockSpec((1, 64), lambda i: (0, 0)),
        ],
        out_specs=pl.BlockSpec((TB, 1), lambda i: (i, 0)),
        out_shape=jax.ShapeDtypeStruct((batch, 1), jnp.float32),
    )(pooled, pooled, W1, b1.reshape(1, -1), W2, b2.reshape(1, -1),
      W3, b3.reshape(1, -1))
    return out.reshape(-1)


def kernel(a, b, emb, W1, b1, W2, b2, W3, b3):
    batch = a.shape[0]
    af = a.astype(jnp.int32).reshape(-1)
    bf = b.astype(jnp.int32).reshape(-1)
    vocab = emb.shape[0]
    packed = _tc_pack(emb.reshape(vocab // 2, 2 * EMBD))
    embp = packed.reshape(vocab, WPR)
    pooled = _sc_pool(embp, (af, bf), 2 * batch)
    return _tc_mlp_cos(pooled, W1, b1, W2, b2, W3, b3, batch)
